# Initial kernel scaffold; baseline (speedup 1.0000x reference)
import jax
import jax.numpy as jnp
from jax.experimental import pallas as pl


def kernel(edge_index, x, W_proj, b_proj, W1, b1, W2, b2):
    raise NotImplementedError("write your pallas kernel here")


# speedup vs baseline: 8.0506x; 8.0506x over previous
import functools

import jax
import jax.numpy as jnp
from jax import lax
from jax.experimental import pallas as pl
from jax.experimental.pallas import tpu as pltpu
from jax.experimental.pallas import tpu_sc as plsc

N = 10000
NPAD = 10240
E = 320000
F = 128
CHUNK = 128
ROWS_PER_TILE = 80
NTILES = 32
EPAD = NTILES * ROWS_PER_TILE * CHUNK
STRIPE = NPAD // 16
DEGW = 128


def _sc_mesh():
    return plsc.VectorSubcoreMesh(
        core_axis_name="c", subcore_axis_name="s", num_cores=2, num_subcores=16
    )


def _deg_body(dst2, ones_hbm, zeros_hbm, out, idxs, ones_v, deg_sh):
    cid = lax.axis_index("c")
    sid = lax.axis_index("s")
    tile = cid * 16 + sid
    base = tile * ROWS_PER_TILE
    pltpu.sync_copy(dst2.at[pl.ds(base, ROWS_PER_TILE)], idxs)
    pltpu.sync_copy(ones_hbm, ones_v)
    stripe = sid * STRIPE
    pltpu.sync_copy(
        zeros_hbm.at[pl.ds(stripe, STRIPE)], deg_sh.at[pl.ds(stripe, STRIPE)]
    )
    plsc.subcore_barrier()

    def body(j, carry):
        pltpu.sync_copy(ones_v, deg_sh.at[idxs.at[j]], add=True)
        return carry

    lax.fori_loop(0, ROWS_PER_TILE, body, 0)
    plsc.subcore_barrier()
    pltpu.sync_copy(
        deg_sh.at[pl.ds(stripe, STRIPE)], out.at[cid, pl.ds(stripe, STRIPE)]
    )


def _edge_body(src2, dst2, table, zeros_hbm, out, isrc, idst, rows, acc_sh):
    cid = lax.axis_index("c")
    sid = lax.axis_index("s")
    tile = cid * 16 + sid
    base = tile * ROWS_PER_TILE
    pltpu.sync_copy(src2.at[pl.ds(base, ROWS_PER_TILE)], isrc)
    pltpu.sync_copy(dst2.at[pl.ds(base, ROWS_PER_TILE)], idst)
    stripe = sid * STRIPE
    pltpu.sync_copy(
        zeros_hbm.at[pl.ds(stripe, STRIPE)], acc_sh.at[pl.ds(stripe, STRIPE)]
    )
    plsc.subcore_barrier()

    def body(j, carry):
        pltpu.sync_copy(table.at[isrc.at[j]], rows)
        pltpu.sync_copy(rows, acc_sh.at[idst.at[j]], add=True)
        return carry

    lax.fori_loop(0, ROWS_PER_TILE, body, 0)
    plsc.subcore_barrier()
    pltpu.sync_copy(
        acc_sh.at[pl.ds(stripe, STRIPE)], out.at[cid, pl.ds(stripe, STRIPE)]
    )


def _deg_call(dst2, ones_hbm, zeros_hbm):
    return pl.kernel(
        _deg_body,
        out_type=jax.ShapeDtypeStruct((2, NPAD, DEGW), jnp.float32),
        mesh=_sc_mesh(),
        scratch_types=[
            pltpu.VMEM((ROWS_PER_TILE, CHUNK), jnp.int32),
            pltpu.VMEM((CHUNK, DEGW), jnp.float32),
            pltpu.VMEM_SHARED((NPAD, DEGW), jnp.float32),
        ],
    )(dst2, ones_hbm, zeros_hbm)


def _edge_call(src2, dst2, table, zeros_hbm):
    return pl.kernel(
        _edge_body,
        out_type=jax.ShapeDtypeStruct((2, NPAD, F), jnp.float32),
        mesh=_sc_mesh(),
        scratch_types=[
            pltpu.VMEM((ROWS_PER_TILE, CHUNK), jnp.int32),
            pltpu.VMEM((ROWS_PER_TILE, CHUNK), jnp.int32),
            pltpu.VMEM((CHUNK, F), jnp.float32),
            pltpu.VMEM_SHARED((NPAD, F), jnp.float32),
        ],
    )(src2, dst2, table, zeros_hbm)


def _tc_pre_body(dp0, dp1, xpad, W1, Wp, bp, dis_o, h1p_o, xproj_o):
    dis = lax.rsqrt(dp0[...] + dp1[...] + 1.0)
    dis_o[...] = dis
    h1 = jnp.dot(xpad[...], W1[...], preferred_element_type=jnp.float32)
    h1p_o[...] = h1 * dis
    xproj_o[...] = (
        jnp.dot(xpad[...], Wp[...], preferred_element_type=jnp.float32) + bp[...]
    )


def _tc_mid_body(a0, a1, h1p, dis, b1, xproj, W2a, W2b, h2p_o):
    z = jnp.maximum(dis[...] * (a0[...] + a1[...] + h1p[...]) + b1[...], 0.0)
    h2 = jnp.dot(z, W2a[...], preferred_element_type=jnp.float32) + jnp.dot(
        xproj[...], W2b[...], preferred_element_type=jnp.float32
    )
    h2p_o[...] = h2 * dis[...]


def _tc_post_body(a0, a1, h2p, dis, b2, out_o):
    out_o[...] = dis[...] * (a0[...] + a1[...] + h2p[...]) + b2[...]


def _tc_pre(dp0, dp1, xpad, W1, Wp, bp):
    return pl.pallas_call(
        _tc_pre_body,
        out_shape=(
            jax.ShapeDtypeStruct((NPAD, 1), jnp.float32),
            jax.ShapeDtypeStruct((NPAD, F), jnp.float32),
            jax.ShapeDtypeStruct((NPAD, 4), jnp.float32),
        ),
    )(dp0, dp1, xpad, W1, Wp, bp)


def _tc_mid(a0, a1, h1p, dis, b1, xproj, W2a, W2b):
    return pl.pallas_call(
        _tc_mid_body,
        out_shape=jax.ShapeDtypeStruct((NPAD, F), jnp.float32),
    )(a0, a1, h1p, dis, b1, xproj, W2a, W2b)


def _tc_post(a0, a1, h2p, dis, b2):
    return pl.pallas_call(
        _tc_post_body,
        out_shape=jax.ShapeDtypeStruct((NPAD, F), jnp.float32),
    )(a0, a1, h2p, dis, b2)


def kernel(edge_index, x, W_proj, b_proj, W1, b1, W2, b2):
    pad = jnp.full((EPAD - E,), NPAD - 1, jnp.int32)
    src2 = jnp.concatenate([edge_index[0], pad]).reshape(EPAD // CHUNK, CHUNK)
    dst2 = jnp.concatenate([edge_index[1], pad]).reshape(EPAD // CHUNK, CHUNK)
    xpad = jnp.pad(x, ((0, NPAD - N), (0, 0)))
    ones_hbm = jnp.ones((CHUNK, DEGW), jnp.float32)
    zeros_deg = jnp.zeros((NPAD, DEGW), jnp.float32)
    zeros_acc = jnp.zeros((NPAD, F), jnp.float32)

    degpart = _deg_call(dst2, ones_hbm, zeros_deg)
    dp = degpart[:, :, 0:1]
    dis, h1p, xproj = _tc_pre(
        dp[0], dp[1], xpad, W1, W_proj, b_proj.reshape(1, 4)
    )
    acc1 = _edge_call(src2, dst2, h1p, zeros_acc)
    h2p = _tc_mid(
        acc1[0], acc1[1], h1p, dis, b1.reshape(1, F), xproj, W2[:F], W2[F:]
    )
    acc2 = _edge_call(src2, dst2, h2p, zeros_acc)
    out = _tc_post(acc2[0], acc2[1], h2p, dis, b2.reshape(1, F))
    return out[:N]


# speedup vs baseline: 8.8406x; 1.0981x over previous
import functools

import jax
import jax.numpy as jnp
from jax import lax
from jax.experimental import pallas as pl
from jax.experimental.pallas import tpu as pltpu
from jax.experimental.pallas import tpu_sc as plsc

N = 10000
NPAD = 10240
E = 320000
F = 128
CHUNK = 128
ROWS_PER_TILE = 80
NTILES = 32
EPAD = NTILES * ROWS_PER_TILE * CHUNK
STRIPE = NPAD // 16
DEGW = 128


def _sc_mesh():
    return plsc.VectorSubcoreMesh(
        core_axis_name="c", subcore_axis_name="s", num_cores=2, num_subcores=16
    )


def _deg_body(dst2, ones_hbm, zeros_hbm, out, idxs, ones_v, deg_sh):
    cid = lax.axis_index("c")
    sid = lax.axis_index("s")
    tile = cid * 16 + sid
    base = tile * ROWS_PER_TILE
    pltpu.sync_copy(dst2.at[pl.ds(base, ROWS_PER_TILE)], idxs)
    pltpu.sync_copy(ones_hbm, ones_v)
    stripe = sid * STRIPE
    pltpu.sync_copy(
        zeros_hbm.at[pl.ds(stripe, STRIPE)], deg_sh.at[pl.ds(stripe, STRIPE)]
    )
    plsc.subcore_barrier()

    def body(j, carry):
        pltpu.sync_copy(ones_v, deg_sh.at[idxs.at[j]], add=True)
        return carry

    lax.fori_loop(0, ROWS_PER_TILE, body, 0)
    plsc.subcore_barrier()
    pltpu.sync_copy(
        deg_sh.at[pl.ds(stripe, STRIPE)], out.at[cid, pl.ds(stripe, STRIPE)]
    )


def _edge_body(src2, dst2, table, zeros_hbm, out, isrc, idst, rows, acc_sh):
    cid = lax.axis_index("c")
    sid = lax.axis_index("s")
    tile = cid * 16 + sid
    base = tile * ROWS_PER_TILE
    pltpu.sync_copy(src2.at[pl.ds(base, ROWS_PER_TILE)], isrc)
    pltpu.sync_copy(dst2.at[pl.ds(base, ROWS_PER_TILE)], idst)
    stripe = sid * STRIPE
    pltpu.sync_copy(
        zeros_hbm.at[pl.ds(stripe, STRIPE)], acc_sh.at[pl.ds(stripe, STRIPE)]
    )
    plsc.subcore_barrier()

    def body(j, carry):
        pltpu.sync_copy(table.at[cid].at[isrc.at[j]], rows)
        pltpu.sync_copy(rows, acc_sh.at[idst.at[j]], add=True)
        return carry

    lax.fori_loop(0, ROWS_PER_TILE, body, 0)
    plsc.subcore_barrier()
    pltpu.sync_copy(
        acc_sh.at[pl.ds(stripe, STRIPE)], out.at[cid, pl.ds(stripe, STRIPE)]
    )


def _deg_call(dst2, ones_hbm, zeros_hbm):
    return pl.kernel(
        _deg_body,
        out_type=jax.ShapeDtypeStruct((2, NPAD, DEGW), jnp.float32),
        mesh=_sc_mesh(),
        scratch_types=[
            pltpu.VMEM((ROWS_PER_TILE, CHUNK), jnp.int32),
            pltpu.VMEM((CHUNK, DEGW), jnp.float32),
            pltpu.VMEM_SHARED((NPAD, DEGW), jnp.float32),
        ],
    )(dst2, ones_hbm, zeros_hbm)


def _edge_call(src2, dst2, table, zeros_hbm):
    tables = jnp.broadcast_to(table, (2, NPAD, F))
    return pl.kernel(
        _edge_body,
        out_type=jax.ShapeDtypeStruct((2, NPAD, F), jnp.float32),
        mesh=_sc_mesh(),
        scratch_types=[
            pltpu.VMEM((ROWS_PER_TILE, CHUNK), jnp.int32),
            pltpu.VMEM((ROWS_PER_TILE, CHUNK), jnp.int32),
            pltpu.VMEM((CHUNK, F), jnp.float32),
            pltpu.VMEM_SHARED((NPAD, F), jnp.float32),
        ],
    )(src2, dst2, tables, zeros_hbm)


def _tc_pre_body(dp0, dp1, xpad, W1, Wp, bp, dis_o, h1p_o, xproj_o):
    dis = lax.rsqrt(dp0[...] + dp1[...] + 1.0)
    dis_o[...] = dis
    h1 = jnp.dot(xpad[...], W1[...], preferred_element_type=jnp.float32)
    h1p_o[...] = h1 * dis
    xproj_o[...] = (
        jnp.dot(xpad[...], Wp[...], preferred_element_type=jnp.float32) + bp[...]
    )


def _tc_mid_body(a0, a1, h1p, dis, b1, xproj, W2a, W2b, h2p_o):
    z = jnp.maximum(dis[...] * (a0[...] + a1[...] + h1p[...]) + b1[...], 0.0)
    h2 = jnp.dot(z, W2a[...], preferred_element_type=jnp.float32) + jnp.dot(
        xproj[...], W2b[...], preferred_element_type=jnp.float32
    )
    h2p_o[...] = h2 * dis[...]


def _tc_post_body(a0, a1, h2p, dis, b2, out_o):
    out_o[...] = dis[...] * (a0[...] + a1[...] + h2p[...]) + b2[...]


def _tc_pre(dp0, dp1, xpad, W1, Wp, bp):
    return pl.pallas_call(
        _tc_pre_body,
        out_shape=(
            jax.ShapeDtypeStruct((NPAD, 1), jnp.float32),
            jax.ShapeDtypeStruct((NPAD, F), jnp.float32),
            jax.ShapeDtypeStruct((NPAD, 4), jnp.float32),
        ),
    )(dp0, dp1, xpad, W1, Wp, bp)


def _tc_mid(a0, a1, h1p, dis, b1, xproj, W2a, W2b):
    return pl.pallas_call(
        _tc_mid_body,
        out_shape=jax.ShapeDtypeStruct((NPAD, F), jnp.float32),
    )(a0, a1, h1p, dis, b1, xproj, W2a, W2b)


def _tc_post(a0, a1, h2p, dis, b2):
    return pl.pallas_call(
        _tc_post_body,
        out_shape=jax.ShapeDtypeStruct((NPAD, F), jnp.float32),
    )(a0, a1, h2p, dis, b2)


def kernel(edge_index, x, W_proj, b_proj, W1, b1, W2, b2):
    pad = jnp.full((EPAD - E,), NPAD - 1, jnp.int32)
    src2 = jnp.concatenate([edge_index[0], pad]).reshape(EPAD // CHUNK, CHUNK)
    dst2 = jnp.concatenate([edge_index[1], pad]).reshape(EPAD // CHUNK, CHUNK)
    xpad = jnp.pad(x, ((0, NPAD - N), (0, 0)))
    ones_hbm = jnp.ones((CHUNK, DEGW), jnp.float32)
    zeros_deg = jnp.zeros((NPAD, DEGW), jnp.float32)
    zeros_acc = jnp.zeros((NPAD, F), jnp.float32)

    degpart = _deg_call(dst2, ones_hbm, zeros_deg)
    dp = degpart[:, :, 0:1]
    dis, h1p, xproj = _tc_pre(
        dp[0], dp[1], xpad, W1, W_proj, b_proj.reshape(1, 4)
    )
    acc1 = _edge_call(src2, dst2, h1p, zeros_acc)
    h2p = _tc_mid(
        acc1[0], acc1[1], h1p, dis, b1.reshape(1, F), xproj, W2[:F], W2[F:]
    )
    acc2 = _edge_call(src2, dst2, h2p, zeros_acc)
    out = _tc_post(acc2[0], acc2[1], h2p, dis, b2.reshape(1, F))
    return out[:N]


# speedup vs baseline: 9.5545x; 1.0808x over previous
import functools

import jax
import jax.numpy as jnp
from jax import lax
from jax.experimental import pallas as pl
from jax.experimental.pallas import tpu as pltpu
from jax.experimental.pallas import tpu_sc as plsc

N = 10000
NPAD = 10240
E = 320000
F = 128
CHUNK = 128
ROWS_PER_TILE = 80
NTILES = 32
EPAD = NTILES * ROWS_PER_TILE * CHUNK
STRIPE = NPAD // 16
HALF = NPAD // 2
ROWS_PER_TILE2 = EPAD // CHUNK // 16
DEGW = 128


def _sc_mesh():
    return plsc.VectorSubcoreMesh(
        core_axis_name="c", subcore_axis_name="s", num_cores=2, num_subcores=16
    )


def _deg_body(dst2, ones_hbm, zeros_hbm, out, idxs, ones_v, deg_sh):
    cid = lax.axis_index("c")
    sid = lax.axis_index("s")
    tile = cid * 16 + sid
    base = tile * ROWS_PER_TILE
    pltpu.sync_copy(dst2.at[pl.ds(base, ROWS_PER_TILE)], idxs)
    pltpu.sync_copy(ones_hbm, ones_v)
    stripe = sid * STRIPE
    pltpu.sync_copy(
        zeros_hbm.at[pl.ds(stripe, STRIPE)], deg_sh.at[pl.ds(stripe, STRIPE)]
    )
    plsc.subcore_barrier()

    def body(j, carry):
        pltpu.sync_copy(ones_v, deg_sh.at[idxs.at[j]], add=True)
        return carry

    lax.fori_loop(0, ROWS_PER_TILE, body, 0)
    plsc.subcore_barrier()
    pltpu.sync_copy(
        deg_sh.at[pl.ds(stripe, STRIPE)], out.at[cid, pl.ds(stripe, STRIPE)]
    )


IBLK = 16
NBLK = ROWS_PER_TILE // IBLK


def _edge_body(
    src2, dst2, table, zeros_hbm, out,
    is_a, is_b, id_a, id_b, rows_a, rows_b, acc_sh,
    sia, sib, sga, sgb, ssa, ssb,
):
    cid = lax.axis_index("c")
    sid = lax.axis_index("s")
    tile = cid * 16 + sid
    base = tile * ROWS_PER_TILE
    stripe = sid * STRIPE
    pltpu.sync_copy(
        zeros_hbm.at[pl.ds(stripe, STRIPE)], acc_sh.at[pl.ds(stripe, STRIPE)]
    )
    plsc.subcore_barrier()

    ibank = [(is_a, id_a, sia), (is_b, id_b, sib)]
    rbuf = [(rows_a, sga, ssa), (rows_b, sgb, ssb)]

    def iload(b):
        bs, bd, sem = ibank[b % 2]
        pltpu.async_copy(src2.at[pl.ds(base + b * IBLK, IBLK)], bs, sem)
        pltpu.async_copy(dst2.at[pl.ds(base + b * IBLK, IBLK)], bd, sem)

    def iwait(b):
        bs, bd, sem = ibank[b % 2]
        pltpu.make_async_copy(src2.at[pl.ds(base + b * IBLK, IBLK)], bs, sem).wait()
        pltpu.make_async_copy(dst2.at[pl.ds(base + b * IBLK, IBLK)], bd, sem).wait()

    def gather(j):
        bs, _, _ = ibank[(j // IBLK) % 2]
        buf, gsem, _ = rbuf[j % 2]
        pltpu.async_copy(table.at[cid].at[bs.at[j % IBLK]], buf, gsem)

    def gwait(j):
        bs, _, _ = ibank[(j // IBLK) % 2]
        buf, gsem, _ = rbuf[j % 2]
        pltpu.make_async_copy(table.at[cid].at[bs.at[j % IBLK]], buf, gsem).wait()

    def scat(j):
        _, bd, _ = ibank[(j // IBLK) % 2]
        buf, _, ssem = rbuf[j % 2]
        pltpu.async_copy(buf, acc_sh.at[bd.at[j % IBLK]], ssem, add=True)

    def swait(j):
        _, bd, _ = ibank[(j // IBLK) % 2]
        buf, _, ssem = rbuf[j % 2]
        pltpu.make_async_copy(buf, acc_sh.at[bd.at[j % IBLK]], ssem).wait()

    iload(0)
    iwait(0)
    gather(0)
    if NBLK > 1:
        iload(1)
    for j in range(ROWS_PER_TILE):
        gwait(j)
        if j + 1 < ROWS_PER_TILE:
            if (j + 1) % IBLK == 0:
                iwait((j + 1) // IBLK)
            gather(j + 1)
        scat(j)
        swait(j)
        if (j + 1) % IBLK == 0 and (j // IBLK) + 2 < NBLK:
            iload((j // IBLK) + 2)
    plsc.subcore_barrier()
    pltpu.sync_copy(
        acc_sh.at[pl.ds(stripe, STRIPE)], out.at[cid, pl.ds(stripe, STRIPE)]
    )


def _deg_call(dst2, ones_hbm, zeros_hbm):
    return pl.kernel(
        _deg_body,
        out_type=jax.ShapeDtypeStruct((2, NPAD, DEGW), jnp.float32),
        mesh=_sc_mesh(),
        scratch_types=[
            pltpu.VMEM((ROWS_PER_TILE, CHUNK), jnp.int32),
            pltpu.VMEM((CHUNK, DEGW), jnp.float32),
            pltpu.VMEM_SHARED((NPAD, DEGW), jnp.float32),
        ],
    )(dst2, ones_hbm, zeros_hbm)


def _edge_call(src2, dst2, table, zeros_hbm):
    tables = jnp.broadcast_to(table, (2, NPAD, F))
    return pl.kernel(
        _edge_body,
        out_type=jax.ShapeDtypeStruct((2, NPAD, F), jnp.float32),
        mesh=_sc_mesh(),
        scratch_types=[
            pltpu.VMEM((IBLK, CHUNK), jnp.int32),
            pltpu.VMEM((IBLK, CHUNK), jnp.int32),
            pltpu.VMEM((IBLK, CHUNK), jnp.int32),
            pltpu.VMEM((IBLK, CHUNK), jnp.int32),
            pltpu.VMEM((CHUNK, F), jnp.float32),
            pltpu.VMEM((CHUNK, F), jnp.float32),
            pltpu.VMEM_SHARED((NPAD, F), jnp.float32),
            pltpu.SemaphoreType.DMA,
            pltpu.SemaphoreType.DMA,
            pltpu.SemaphoreType.DMA,
            pltpu.SemaphoreType.DMA,
            pltpu.SemaphoreType.DMA,
            pltpu.SemaphoreType.DMA,
        ],
    )(src2, dst2, tables, zeros_hbm)


def _tc_pre_body(dp0, dp1, xpad, W1, Wp, bp, dis_o, h1p_o, xproj_o):
    dis = lax.rsqrt(dp0[...] + dp1[...] + 1.0)
    dis_o[...] = dis
    h1 = jnp.dot(xpad[...], W1[...], preferred_element_type=jnp.float32)
    h1p_o[...] = h1 * dis
    xproj_o[...] = (
        jnp.dot(xpad[...], Wp[...], preferred_element_type=jnp.float32) + bp[...]
    )


def _tc_mid_body(a0, a1, h1p, dis, b1, xproj, W2a, W2b, h2p_o):
    z = jnp.maximum(dis[...] * (a0[...] + a1[...] + h1p[...]) + b1[...], 0.0)
    h2 = jnp.dot(z, W2a[...], preferred_element_type=jnp.float32) + jnp.dot(
        xproj[...], W2b[...], preferred_element_type=jnp.float32
    )
    h2p_o[...] = h2 * dis[...]


def _tc_post_body(a0, a1, h2p, dis, b2, out_o):
    out_o[...] = dis[...] * (a0[...] + a1[...] + h2p[...]) + b2[...]


def _tc_pre(dp0, dp1, xpad, W1, Wp, bp):
    return pl.pallas_call(
        _tc_pre_body,
        out_shape=(
            jax.ShapeDtypeStruct((NPAD, 1), jnp.float32),
            jax.ShapeDtypeStruct((NPAD, F), jnp.float32),
            jax.ShapeDtypeStruct((NPAD, 4), jnp.float32),
        ),
    )(dp0, dp1, xpad, W1, Wp, bp)


def _tc_mid(a0, a1, h1p, dis, b1, xproj, W2a, W2b):
    return pl.pallas_call(
        _tc_mid_body,
        out_shape=jax.ShapeDtypeStruct((NPAD, F), jnp.float32),
    )(a0, a1, h1p, dis, b1, xproj, W2a, W2b)


def _tc_post(a0, a1, h2p, dis, b2):
    return pl.pallas_call(
        _tc_post_body,
        out_shape=jax.ShapeDtypeStruct((NPAD, F), jnp.float32),
    )(a0, a1, h2p, dis, b2)


def kernel(edge_index, x, W_proj, b_proj, W1, b1, W2, b2):
    pad = jnp.full((EPAD - E,), NPAD - 1, jnp.int32)
    src2 = jnp.concatenate([edge_index[0], pad]).reshape(EPAD // CHUNK, CHUNK)
    dst2 = jnp.concatenate([edge_index[1], pad]).reshape(EPAD // CHUNK, CHUNK)
    xpad = jnp.pad(x, ((0, NPAD - N), (0, 0)))
    ones_hbm = jnp.ones((CHUNK, DEGW), jnp.float32)
    zeros_deg = jnp.zeros((NPAD, DEGW), jnp.float32)
    zeros_acc = jnp.zeros((NPAD, F), jnp.float32)

    degpart = _deg_call(dst2, ones_hbm, zeros_deg)
    dp = degpart[:, :, 0:1]
    dis, h1p, xproj = _tc_pre(
        dp[0], dp[1], xpad, W1, W_proj, b_proj.reshape(1, 4)
    )
    acc1 = _edge_call(src2, dst2, h1p, zeros_acc)
    h2p = _tc_mid(
        acc1[0], acc1[1], h1p, dis, b1.reshape(1, F), xproj, W2[:F], W2[F:]
    )
    acc2 = _edge_call(src2, dst2, h2p, zeros_acc)
    out = _tc_post(acc2[0], acc2[1], h2p, dis, b2.reshape(1, F))
    return out[:N]


# speedup vs baseline: 18.1091x; 1.8954x over previous
import jax
import jax.numpy as jnp
from jax import lax
from jax.experimental import pallas as pl
from jax.experimental.pallas import tpu as pltpu
from jax.experimental.pallas import tpu_sc as plsc

N = 10000
NPAD = 10240
E = 320000
F = 128
FH = 64
CHUNK = 128
ROWS_PER_TILE = 80
NTILES = 32
EPAD = NTILES * ROWS_PER_TILE * CHUNK
STRIPE = NPAD // 16
DEGW = 64
IBLK = 16
NBLK = ROWS_PER_TILE // IBLK

_CP = pltpu.CompilerParams(use_tc_tiling_on_sc=False)


def _sc_mesh():
    return plsc.VectorSubcoreMesh(
        core_axis_name="c", subcore_axis_name="s", num_cores=2, num_subcores=16
    )


def _deg_body(dst2, ones_hbm, zeros_hbm, out, idxs, ones_v, deg_sh):
    cid = lax.axis_index("c")
    sid = lax.axis_index("s")
    tile = cid * 16 + sid
    base = tile * ROWS_PER_TILE
    pltpu.sync_copy(dst2.at[pl.ds(base, ROWS_PER_TILE)], idxs)
    pltpu.sync_copy(ones_hbm, ones_v)
    stripe = sid * STRIPE
    pltpu.sync_copy(
        zeros_hbm.at[pl.ds(stripe, STRIPE)], deg_sh.at[pl.ds(stripe, STRIPE)]
    )
    plsc.subcore_barrier()

    def body(j, carry):
        pltpu.sync_copy(ones_v, deg_sh.at[idxs.at[j]], add=True)
        return carry

    lax.fori_loop(0, ROWS_PER_TILE, body, 0)
    plsc.subcore_barrier()
    pltpu.sync_copy(
        deg_sh.at[pl.ds(stripe, STRIPE)], out.at[cid, pl.ds(stripe, STRIPE)]
    )


def _deg_call(dst2, ones_hbm, zeros_hbm):
    return pl.kernel(
        _deg_body,
        out_type=jax.ShapeDtypeStruct((2, NPAD, DEGW), jnp.float32),
        mesh=_sc_mesh(),
        compiler_params=_CP,
        scratch_types=[
            pltpu.VMEM((ROWS_PER_TILE, CHUNK), jnp.int32),
            pltpu.VMEM((CHUNK, DEGW), jnp.float32),
            pltpu.VMEM_SHARED((NPAD, DEGW), jnp.float32),
        ],
    )(dst2, ones_hbm, zeros_hbm)


def _edge_body(
    src2, dst2, tab_lo, tab_hi, zeros_hbm, out_lo, out_hi,
    is_a, is_b, id_a, id_b, rows_a, rows_b, acc_sh, tab_sh,
    sia, sib, sga, sgb, ssa, ssb,
):
    cid = lax.axis_index("c")
    sid = lax.axis_index("s")
    tile = cid * 16 + sid
    base = tile * ROWS_PER_TILE
    stripe = sid * STRIPE

    ibank = [(is_a, id_a, sia), (is_b, id_b, sib)]
    rbuf = [(rows_a, sga, ssa), (rows_b, sgb, ssb)]

    def iload(b):
        bs, bd, sem = ibank[b % 2]
        pltpu.async_copy(src2.at[pl.ds(base + b * IBLK, IBLK)], bs, sem)
        pltpu.async_copy(dst2.at[pl.ds(base + b * IBLK, IBLK)], bd, sem)

    def iwait(b):
        bs, bd, sem = ibank[b % 2]
        pltpu.make_async_copy(src2.at[pl.ds(base + b * IBLK, IBLK)], bs, sem).wait()
        pltpu.make_async_copy(dst2.at[pl.ds(base + b * IBLK, IBLK)], bd, sem).wait()

    def gather(j):
        bs, _, _ = ibank[(j // IBLK) % 2]
        buf, gsem, _ = rbuf[j % 2]
        pltpu.async_copy(tab_sh.at[bs.at[j % IBLK]], buf, gsem)

    def gwait(j):
        bs, _, _ = ibank[(j // IBLK) % 2]
        buf, gsem, _ = rbuf[j % 2]
        pltpu.make_async_copy(tab_sh.at[bs.at[j % IBLK]], buf, gsem).wait()

    def scat(j):
        _, bd, _ = ibank[(j // IBLK) % 2]
        buf, _, ssem = rbuf[j % 2]
        pltpu.async_copy(buf, acc_sh.at[bd.at[j % IBLK]], ssem, add=True)

    def swait(j):
        _, bd, _ = ibank[(j // IBLK) % 2]
        buf, _, ssem = rbuf[j % 2]
        pltpu.make_async_copy(buf, acc_sh.at[bd.at[j % IBLK]], ssem).wait()

    for tab_hbm, out_hbm in ((tab_lo, out_lo), (tab_hi, out_hi)):
        pltpu.sync_copy(
            tab_hbm.at[pl.ds(stripe, STRIPE)], tab_sh.at[pl.ds(stripe, STRIPE)]
        )
        pltpu.sync_copy(
            zeros_hbm.at[pl.ds(stripe, STRIPE)], acc_sh.at[pl.ds(stripe, STRIPE)]
        )
        plsc.subcore_barrier()

        iload(0)
        iwait(0)
        gather(0)
        if NBLK > 1:
            iload(1)
        for j in range(ROWS_PER_TILE):
            gwait(j)
            if j + 1 < ROWS_PER_TILE:
                if (j + 1) % IBLK == 0:
                    iwait((j + 1) // IBLK)
                gather(j + 1)
            scat(j)
            swait(j)
            if (j + 1) % IBLK == 0 and (j // IBLK) + 2 < NBLK:
                iload((j // IBLK) + 2)
        plsc.subcore_barrier()
        pltpu.sync_copy(
            acc_sh.at[pl.ds(stripe, STRIPE)], out_hbm.at[cid, pl.ds(stripe, STRIPE)]
        )


def _edge_call(src2, dst2, tab_lo, tab_hi, zeros_hbm):
    return pl.kernel(
        _edge_body,
        out_type=(
            jax.ShapeDtypeStruct((2, NPAD, FH), jnp.float32),
            jax.ShapeDtypeStruct((2, NPAD, FH), jnp.float32),
        ),
        mesh=_sc_mesh(),
        compiler_params=_CP,
        scratch_types=[
            pltpu.VMEM((IBLK, CHUNK), jnp.int32),
            pltpu.VMEM((IBLK, CHUNK), jnp.int32),
            pltpu.VMEM((IBLK, CHUNK), jnp.int32),
            pltpu.VMEM((IBLK, CHUNK), jnp.int32),
            pltpu.VMEM((CHUNK, FH), jnp.float32),
            pltpu.VMEM((CHUNK, FH), jnp.float32),
            pltpu.VMEM_SHARED((NPAD, FH), jnp.float32),
            pltpu.VMEM_SHARED((NPAD, FH), jnp.float32),
            pltpu.SemaphoreType.DMA,
            pltpu.SemaphoreType.DMA,
            pltpu.SemaphoreType.DMA,
            pltpu.SemaphoreType.DMA,
            pltpu.SemaphoreType.DMA,
            pltpu.SemaphoreType.DMA,
        ],
    )(src2, dst2, tab_lo, tab_hi, zeros_hbm)


def _tc_mm_body(xpad, W1, Wp, bp, h1_o, xproj_o):
    h1_o[...] = jnp.dot(xpad[...], W1[...], preferred_element_type=jnp.float32)
    xproj_o[...] = (
        jnp.dot(xpad[...], Wp[...], preferred_element_type=jnp.float32) + bp[...]
    )


def _tc_scale_body(dp0, dp1, h1, dis_o, h1p_o):
    dis = lax.rsqrt(dp0[...] + dp1[...] + 1.0)
    dis_o[...] = dis
    h1p_o[...] = h1[...] * dis


def _tc_mid_body(al0, al1, ah0, ah1, h1p, dis, b1, xproj, W2a, W2b, h2p_o):
    accsum = jnp.concatenate(
        [al0[...] + al1[...], ah0[...] + ah1[...]], axis=1
    )
    z = jnp.maximum(dis[...] * (accsum + h1p[...]) + b1[...], 0.0)
    h2 = jnp.dot(z, W2a[...], preferred_element_type=jnp.float32) + jnp.dot(
        xproj[...], W2b[...], preferred_element_type=jnp.float32
    )
    h2p_o[...] = h2 * dis[...]


def _tc_post_body(al0, al1, ah0, ah1, h2p, dis, b2, out_o):
    accsum = jnp.concatenate(
        [al0[...] + al1[...], ah0[...] + ah1[...]], axis=1
    )
    out_o[...] = dis[...] * (accsum + h2p[...]) + b2[...]


def _tc_mm(xpad, W1, Wp, bp):
    return pl.pallas_call(
        _tc_mm_body,
        out_shape=(
            jax.ShapeDtypeStruct((NPAD, F), jnp.float32),
            jax.ShapeDtypeStruct((NPAD, 4), jnp.float32),
        ),
    )(xpad, W1, Wp, bp)


def _tc_scale(dp0, dp1, h1):
    return pl.pallas_call(
        _tc_scale_body,
        out_shape=(
            jax.ShapeDtypeStruct((NPAD, 1), jnp.float32),
            jax.ShapeDtypeStruct((NPAD, F), jnp.float32),
        ),
    )(dp0, dp1, h1)


def _tc_mid(al0, al1, ah0, ah1, h1p, dis, b1, xproj, W2a, W2b):
    return pl.pallas_call(
        _tc_mid_body,
        out_shape=jax.ShapeDtypeStruct((NPAD, F), jnp.float32),
    )(al0, al1, ah0, ah1, h1p, dis, b1, xproj, W2a, W2b)


def _tc_post(al0, al1, ah0, ah1, h2p, dis, b2):
    return pl.pallas_call(
        _tc_post_body,
        out_shape=jax.ShapeDtypeStruct((NPAD, F), jnp.float32),
    )(al0, al1, ah0, ah1, h2p, dis, b2)


def kernel(edge_index, x, W_proj, b_proj, W1, b1, W2, b2):
    pad = jnp.full((EPAD - E,), NPAD - 1, jnp.int32)
    src2 = jnp.concatenate([edge_index[0], pad]).reshape(EPAD // CHUNK, CHUNK)
    dst2 = jnp.concatenate([edge_index[1], pad]).reshape(EPAD // CHUNK, CHUNK)
    xpad = jnp.pad(x, ((0, NPAD - N), (0, 0)))
    ones_hbm = jnp.ones((CHUNK, DEGW), jnp.float32)
    zeros_deg = jnp.zeros((NPAD, DEGW), jnp.float32)
    zeros_acc = jnp.zeros((NPAD, FH), jnp.float32)

    degpart = _deg_call(dst2, ones_hbm, zeros_deg)
    h1, xproj = _tc_mm(xpad, W1, W_proj, b_proj.reshape(1, 4))
    dp = degpart[:, :, 0:1]
    dis, h1p = _tc_scale(dp[0], dp[1], h1)
    acc1lo, acc1hi = _edge_call(
        src2, dst2, h1p[:, :FH], h1p[:, FH:], zeros_acc
    )
    h2p = _tc_mid(
        acc1lo[0], acc1lo[1], acc1hi[0], acc1hi[1],
        h1p, dis, b1.reshape(1, F), xproj, W2[:F], W2[F:],
    )
    acc2lo, acc2hi = _edge_call(
        src2, dst2, h2p[:, :FH], h2p[:, FH:], zeros_acc
    )
    out = _tc_post(
        acc2lo[0], acc2lo[1], acc2hi[0], acc2hi[1], h2p, dis, b2.reshape(1, F)
    )
    return out[:N]


# speedup vs baseline: 19.0625x; 1.0526x over previous
import jax
import jax.numpy as jnp
from jax import lax
from jax.experimental import pallas as pl
from jax.experimental.pallas import tpu as pltpu
from jax.experimental.pallas import tpu_sc as plsc

N = 10000
NPAD = 10240
E = 320000
F = 128
FH = 64
CHUNK = 128
ROWS_PER_TILE = 80
NTILES = 32
EPAD = NTILES * ROWS_PER_TILE * CHUNK
STRIPE = NPAD // 16
DEGW = 64
IBLK = 16
NBLK = ROWS_PER_TILE // IBLK

_CP = pltpu.CompilerParams(use_tc_tiling_on_sc=False)


def _sc_mesh():
    return plsc.VectorSubcoreMesh(
        core_axis_name="c", subcore_axis_name="s", num_cores=2, num_subcores=16
    )


def _deg_body(dst2, ones_hbm, zeros_hbm, out, idxs, ones_v, deg_sh, dsem):
    cid = lax.axis_index("c")
    sid = lax.axis_index("s")
    tile = cid * 16 + sid
    base = tile * ROWS_PER_TILE
    pltpu.sync_copy(dst2.at[pl.ds(base, ROWS_PER_TILE)], idxs)
    pltpu.sync_copy(ones_hbm, ones_v)
    stripe = sid * STRIPE
    pltpu.sync_copy(
        zeros_hbm.at[pl.ds(stripe, STRIPE)], deg_sh.at[pl.ds(stripe, STRIPE)]
    )
    plsc.subcore_barrier()

    def dscat(j):
        pltpu.async_copy(ones_v, deg_sh.at[idxs.at[j]], dsem, add=True)

    def dwait(j):
        pltpu.make_async_copy(ones_v, deg_sh.at[idxs.at[j]], dsem).wait()

    for j in range(ROWS_PER_TILE):
        dscat(j)
        if j >= 8:
            dwait(j - 8)
    for j in range(ROWS_PER_TILE - 8, ROWS_PER_TILE):
        dwait(j)
    plsc.subcore_barrier()
    pltpu.sync_copy(
        deg_sh.at[pl.ds(stripe, STRIPE)], out.at[cid, pl.ds(stripe, STRIPE)]
    )


def _deg_call(dst2, ones_hbm, zeros_hbm):
    return pl.kernel(
        _deg_body,
        out_type=jax.ShapeDtypeStruct((2, NPAD, DEGW), jnp.float32),
        mesh=_sc_mesh(),
        compiler_params=_CP,
        scratch_types=[
            pltpu.VMEM((ROWS_PER_TILE, CHUNK), jnp.int32),
            pltpu.VMEM((CHUNK, DEGW), jnp.float32),
            pltpu.VMEM_SHARED((NPAD, DEGW), jnp.float32),
            pltpu.SemaphoreType.DMA,
        ],
    )(dst2, ones_hbm, zeros_hbm)


NRB = 4


def _edge_body(
    src2, dst2, tab_lo, tab_hi, zeros_hbm, out_lo, out_hi,
    is_a, is_b, id_a, id_b, rows_bufs, acc_sh, tab_sh,
    sia, sib, gsems, ssems,
):
    cid = lax.axis_index("c")
    sid = lax.axis_index("s")
    tile = cid * 16 + sid
    base = tile * ROWS_PER_TILE
    stripe = sid * STRIPE

    ibank = [(is_a, id_a, sia), (is_b, id_b, sib)]
    rbuf = [(rows_bufs[k], gsems[k], ssems[k]) for k in range(NRB)]

    def iload(b):
        bs, bd, sem = ibank[b % 2]
        pltpu.async_copy(src2.at[pl.ds(base + b * IBLK, IBLK)], bs, sem)
        pltpu.async_copy(dst2.at[pl.ds(base + b * IBLK, IBLK)], bd, sem)

    def iwait(b):
        bs, bd, sem = ibank[b % 2]
        pltpu.make_async_copy(src2.at[pl.ds(base + b * IBLK, IBLK)], bs, sem).wait()
        pltpu.make_async_copy(dst2.at[pl.ds(base + b * IBLK, IBLK)], bd, sem).wait()

    def gather(j):
        bs, _, _ = ibank[(j // IBLK) % 2]
        buf, gsem, _ = rbuf[j % NRB]
        pltpu.async_copy(tab_sh.at[bs.at[j % IBLK]], buf, gsem)

    def gwait(j):
        bs, _, _ = ibank[(j // IBLK) % 2]
        buf, gsem, _ = rbuf[j % NRB]
        pltpu.make_async_copy(tab_sh.at[bs.at[j % IBLK]], buf, gsem).wait()

    def scat(j):
        _, bd, _ = ibank[(j // IBLK) % 2]
        buf, _, ssem = rbuf[j % NRB]
        pltpu.async_copy(buf, acc_sh.at[bd.at[j % IBLK]], ssem, add=True)

    def swait(j):
        _, bd, _ = ibank[(j // IBLK) % 2]
        buf, _, ssem = rbuf[j % NRB]
        pltpu.make_async_copy(buf, acc_sh.at[bd.at[j % IBLK]], ssem).wait()

    for tab_hbm, out_hbm in ((tab_lo, out_lo), (tab_hi, out_hi)):
        pltpu.sync_copy(
            tab_hbm.at[pl.ds(stripe, STRIPE)], tab_sh.at[pl.ds(stripe, STRIPE)]
        )
        pltpu.sync_copy(
            zeros_hbm.at[pl.ds(stripe, STRIPE)], acc_sh.at[pl.ds(stripe, STRIPE)]
        )
        plsc.subcore_barrier()

        iload(0)
        iwait(0)
        for j in range(NRB - 1):
            gather(j)
        for j in range(ROWS_PER_TILE):
            gwait(j)
            scat(j)
            nj = j + NRB - 1
            if nj < ROWS_PER_TILE:
                if nj % IBLK == 0:
                    iwait(nj // IBLK)
                if nj - NRB >= 0:
                    swait(nj - NRB)
                gather(nj)
                if nj % IBLK == 3 and (nj // IBLK) + 1 < NBLK:
                    iload((nj // IBLK) + 1)
        for j in range(ROWS_PER_TILE - NRB, ROWS_PER_TILE):
            swait(j)
        plsc.subcore_barrier()
        pltpu.sync_copy(
            acc_sh.at[pl.ds(stripe, STRIPE)], out_hbm.at[cid, pl.ds(stripe, STRIPE)]
        )


def _edge_call(src2, dst2, tab_lo, tab_hi, zeros_hbm):
    return pl.kernel(
        _edge_body,
        out_type=(
            jax.ShapeDtypeStruct((2, NPAD, FH), jnp.float32),
            jax.ShapeDtypeStruct((2, NPAD, FH), jnp.float32),
        ),
        mesh=_sc_mesh(),
        compiler_params=_CP,
        scratch_types=[
            pltpu.VMEM((IBLK, CHUNK), jnp.int32),
            pltpu.VMEM((IBLK, CHUNK), jnp.int32),
            pltpu.VMEM((IBLK, CHUNK), jnp.int32),
            pltpu.VMEM((IBLK, CHUNK), jnp.int32),
            [pltpu.VMEM((CHUNK, FH), jnp.float32) for _ in range(NRB)],
            pltpu.VMEM_SHARED((NPAD, FH), jnp.float32),
            pltpu.VMEM_SHARED((NPAD, FH), jnp.float32),
            pltpu.SemaphoreType.DMA,
            pltpu.SemaphoreType.DMA,
            [pltpu.SemaphoreType.DMA for _ in range(NRB)],
            [pltpu.SemaphoreType.DMA for _ in range(NRB)],
        ],
    )(src2, dst2, tab_lo, tab_hi, zeros_hbm)


def _tc_mm_body(xpad, W1, Wp, bp, h1_o, xproj_o):
    h1_o[...] = jnp.dot(xpad[...], W1[...], preferred_element_type=jnp.float32)
    xproj_o[...] = (
        jnp.dot(xpad[...], Wp[...], preferred_element_type=jnp.float32) + bp[...]
    )


def _tc_scale_body(dp0, dp1, h1, dis_o, h1p_o):
    dis = lax.rsqrt(dp0[...] + dp1[...] + 1.0)
    dis_o[...] = dis
    h1p_o[...] = h1[...] * dis


def _tc_mid_body(al0, al1, ah0, ah1, h1p, dis, b1, xproj, W2a, W2b, h2p_o):
    accsum = jnp.concatenate(
        [al0[...] + al1[...], ah0[...] + ah1[...]], axis=1
    )
    z = jnp.maximum(dis[...] * (accsum + h1p[...]) + b1[...], 0.0)
    h2 = jnp.dot(z, W2a[...], preferred_element_type=jnp.float32) + jnp.dot(
        xproj[...], W2b[...], preferred_element_type=jnp.float32
    )
    h2p_o[...] = h2 * dis[...]


def _tc_post_body(al0, al1, ah0, ah1, h2p, dis, b2, out_o):
    accsum = jnp.concatenate(
        [al0[...] + al1[...], ah0[...] + ah1[...]], axis=1
    )
    out_o[...] = dis[...] * (accsum + h2p[...]) + b2[...]


def _tc_mm(xpad, W1, Wp, bp):
    return pl.pallas_call(
        _tc_mm_body,
        out_shape=(
            jax.ShapeDtypeStruct((NPAD, F), jnp.float32),
            jax.ShapeDtypeStruct((NPAD, 4), jnp.float32),
        ),
    )(xpad, W1, Wp, bp)


def _tc_scale(dp0, dp1, h1):
    return pl.pallas_call(
        _tc_scale_body,
        out_shape=(
            jax.ShapeDtypeStruct((NPAD, 1), jnp.float32),
            jax.ShapeDtypeStruct((NPAD, F), jnp.float32),
        ),
    )(dp0, dp1, h1)


def _tc_mid(al0, al1, ah0, ah1, h1p, dis, b1, xproj, W2a, W2b):
    return pl.pallas_call(
        _tc_mid_body,
        out_shape=jax.ShapeDtypeStruct((NPAD, F), jnp.float32),
    )(al0, al1, ah0, ah1, h1p, dis, b1, xproj, W2a, W2b)


def _tc_post(al0, al1, ah0, ah1, h2p, dis, b2):
    return pl.pallas_call(
        _tc_post_body,
        out_shape=jax.ShapeDtypeStruct((NPAD, F), jnp.float32),
    )(al0, al1, ah0, ah1, h2p, dis, b2)


def kernel(edge_index, x, W_proj, b_proj, W1, b1, W2, b2):
    pad = jnp.full((EPAD - E,), NPAD - 1, jnp.int32)
    src2 = jnp.concatenate([edge_index[0], pad]).reshape(EPAD // CHUNK, CHUNK)
    dst2 = jnp.concatenate([edge_index[1], pad]).reshape(EPAD // CHUNK, CHUNK)
    xpad = jnp.pad(x, ((0, NPAD - N), (0, 0)))
    ones_hbm = jnp.ones((CHUNK, DEGW), jnp.float32)
    zeros_deg = jnp.zeros((NPAD, DEGW), jnp.float32)
    zeros_acc = jnp.zeros((NPAD, FH), jnp.float32)

    degpart = _deg_call(dst2, ones_hbm, zeros_deg)
    h1, xproj = _tc_mm(xpad, W1, W_proj, b_proj.reshape(1, 4))
    dp = degpart[:, :, 0:1]
    dis, h1p = _tc_scale(dp[0], dp[1], h1)
    acc1lo, acc1hi = _edge_call(
        src2, dst2, h1p[:, :FH], h1p[:, FH:], zeros_acc
    )
    h2p = _tc_mid(
        acc1lo[0], acc1lo[1], acc1hi[0], acc1hi[1],
        h1p, dis, b1.reshape(1, F), xproj, W2[:F], W2[F:],
    )
    acc2lo, acc2hi = _edge_call(
        src2, dst2, h2p[:, :FH], h2p[:, FH:], zeros_acc
    )
    out = _tc_post(
        acc2lo[0], acc2lo[1], acc2hi[0], acc2hi[1], h2p, dis, b2.reshape(1, F)
    )
    return out[:N]


# speedup vs baseline: 20.6909x; 1.0854x over previous
import jax
import jax.numpy as jnp
from jax import lax
from jax.experimental import pallas as pl
from jax.experimental.pallas import tpu as pltpu
from jax.experimental.pallas import tpu_sc as plsc

N = 10000
NPAD = 10240
E = 320000
F = 128
FH = 64
CHUNK = 128
ROWS_PER_TILE = 80
NTILES = 32
EPAD = NTILES * ROWS_PER_TILE * CHUNK
STRIPE = NPAD // 16
DEGW = 64
IBLK = 16
NBLK = ROWS_PER_TILE // IBLK

_CP = pltpu.CompilerParams(use_tc_tiling_on_sc=False)


def _sc_mesh():
    return plsc.VectorSubcoreMesh(
        core_axis_name="c", subcore_axis_name="s", num_cores=2, num_subcores=16
    )


def _deg_body(dst2, ones_hbm, zeros_hbm, out, idxs, ones_v, deg_sh, dsem):
    cid = lax.axis_index("c")
    sid = lax.axis_index("s")
    tile = cid * 16 + sid
    base = tile * ROWS_PER_TILE
    pltpu.sync_copy(dst2.at[pl.ds(base, ROWS_PER_TILE)], idxs)
    pltpu.sync_copy(ones_hbm, ones_v)
    stripe = sid * STRIPE
    pltpu.sync_copy(
        zeros_hbm.at[pl.ds(stripe, STRIPE)], deg_sh.at[pl.ds(stripe, STRIPE)]
    )
    plsc.subcore_barrier()

    def dscat(j):
        pltpu.async_copy(ones_v, deg_sh.at[idxs.at[j]], dsem, add=True)

    def dwait(j):
        pltpu.make_async_copy(ones_v, deg_sh.at[idxs.at[j]], dsem).wait()

    for j in range(ROWS_PER_TILE):
        dscat(j)
        if j >= 8:
            dwait(j - 8)
    for j in range(ROWS_PER_TILE - 8, ROWS_PER_TILE):
        dwait(j)
    plsc.subcore_barrier()
    pltpu.sync_copy(
        deg_sh.at[pl.ds(stripe, STRIPE)], out.at[cid, pl.ds(stripe, STRIPE)]
    )


def _deg_call(dst2, ones_hbm, zeros_hbm):
    return pl.kernel(
        _deg_body,
        out_type=jax.ShapeDtypeStruct((2, NPAD, DEGW), jnp.float32),
        mesh=_sc_mesh(),
        compiler_params=_CP,
        scratch_types=[
            pltpu.VMEM((ROWS_PER_TILE, CHUNK), jnp.int32),
            pltpu.VMEM((CHUNK, DEGW), jnp.float32),
            pltpu.VMEM_SHARED((NPAD, DEGW), jnp.float32),
            pltpu.SemaphoreType.DMA,
        ],
    )(dst2, ones_hbm, zeros_hbm)


NRB = 4


def _edge_body(
    src2, dst2, tab_lo, tab_hi, zeros_hbm, out_lo, out_hi,
    is_a, is_b, id_a, id_b, rows_bufs, acc_sh, tab_sh,
    sia, sib, gsems, ssems,
):
    cid = lax.axis_index("c")
    sid = lax.axis_index("s")
    tile = cid * 16 + sid
    base = tile * ROWS_PER_TILE
    stripe = sid * STRIPE

    ibank = [(is_a, id_a, sia), (is_b, id_b, sib)]
    rbuf = [(rows_bufs[k], gsems[k], ssems[k]) for k in range(NRB)]

    def iload(b):
        bs, bd, sem = ibank[b % 2]
        pltpu.async_copy(src2.at[pl.ds(base + b * IBLK, IBLK)], bs, sem)
        pltpu.async_copy(dst2.at[pl.ds(base + b * IBLK, IBLK)], bd, sem)

    def iwait(b):
        bs, bd, sem = ibank[b % 2]
        pltpu.make_async_copy(src2.at[pl.ds(base + b * IBLK, IBLK)], bs, sem).wait()
        pltpu.make_async_copy(dst2.at[pl.ds(base + b * IBLK, IBLK)], bd, sem).wait()

    def gather(j):
        bs, _, _ = ibank[(j // IBLK) % 2]
        buf, gsem, _ = rbuf[j % NRB]
        pltpu.async_copy(tab_sh.at[bs.at[j % IBLK]], buf, gsem)

    def gwait(j):
        bs, _, _ = ibank[(j // IBLK) % 2]
        buf, gsem, _ = rbuf[j % NRB]
        pltpu.make_async_copy(tab_sh.at[bs.at[j % IBLK]], buf, gsem).wait()

    def scat(j):
        _, bd, _ = ibank[(j // IBLK) % 2]
        buf, _, ssem = rbuf[j % NRB]
        pltpu.async_copy(buf, acc_sh.at[bd.at[j % IBLK]], ssem, add=True)

    def swait(j):
        _, bd, _ = ibank[(j // IBLK) % 2]
        buf, _, ssem = rbuf[j % NRB]
        pltpu.make_async_copy(buf, acc_sh.at[bd.at[j % IBLK]], ssem).wait()

    for tab_hbm, out_hbm in ((tab_lo, out_lo), (tab_hi, out_hi)):
        pltpu.sync_copy(
            tab_hbm.at[pl.ds(stripe, STRIPE)], tab_sh.at[pl.ds(stripe, STRIPE)]
        )
        pltpu.sync_copy(
            zeros_hbm.at[pl.ds(stripe, STRIPE)], acc_sh.at[pl.ds(stripe, STRIPE)]
        )
        plsc.subcore_barrier()

        iload(0)
        iwait(0)
        for j in range(NRB - 1):
            gather(j)
        for j in range(ROWS_PER_TILE):
            gwait(j)
            scat(j)
            nj = j + NRB - 1
            if nj < ROWS_PER_TILE:
                if nj % IBLK == 0:
                    iwait(nj // IBLK)
                if nj - NRB >= 0:
                    swait(nj - NRB)
                gather(nj)
                if nj % IBLK == 3 and (nj // IBLK) + 1 < NBLK:
                    iload((nj // IBLK) + 1)
        for j in range(ROWS_PER_TILE - NRB, ROWS_PER_TILE):
            swait(j)
        plsc.subcore_barrier()
        pltpu.sync_copy(
            acc_sh.at[pl.ds(stripe, STRIPE)], out_hbm.at[cid, pl.ds(stripe, STRIPE)]
        )


def _edge_call(src2, dst2, tab_lo, tab_hi, zeros_hbm):
    return pl.kernel(
        _edge_body,
        out_type=(
            jax.ShapeDtypeStruct((2, NPAD, FH), jnp.float32),
            jax.ShapeDtypeStruct((2, NPAD, FH), jnp.float32),
        ),
        mesh=_sc_mesh(),
        compiler_params=_CP,
        scratch_types=[
            pltpu.VMEM((IBLK, CHUNK), jnp.int32),
            pltpu.VMEM((IBLK, CHUNK), jnp.int32),
            pltpu.VMEM((IBLK, CHUNK), jnp.int32),
            pltpu.VMEM((IBLK, CHUNK), jnp.int32),
            [pltpu.VMEM((CHUNK, FH), jnp.float32) for _ in range(NRB)],
            pltpu.VMEM_SHARED((NPAD, FH), jnp.float32),
            pltpu.VMEM_SHARED((NPAD, FH), jnp.float32),
            pltpu.SemaphoreType.DMA,
            pltpu.SemaphoreType.DMA,
            [pltpu.SemaphoreType.DMA for _ in range(NRB)],
            [pltpu.SemaphoreType.DMA for _ in range(NRB)],
        ],
    )(src2, dst2, tab_lo, tab_hi, zeros_hbm)


def _tc_mm_body(xpad, W1, Wp, bp, h1_o, xproj_o):
    h1_o[...] = jnp.dot(xpad[...], W1[...], preferred_element_type=jnp.float32)
    xproj_o[...] = (
        jnp.dot(xpad[...], Wp[...], preferred_element_type=jnp.float32) + bp[...]
    )


def _tc_scale_body(degpart, h1, dis_o, lo_o, hi_o):
    dp = degpart[0, :, 0:1] + degpart[1, :, 0:1]
    dis = lax.rsqrt(dp + 1.0)
    dis_o[...] = dis
    h1p = h1[...] * dis
    lo_o[...] = h1p[:, :FH]
    hi_o[...] = h1p[:, FH:]


def _tc_mid_body(acclo, acchi, h1lo, h1hi, dis, b1, xproj, W2a, W2b, lo_o, hi_o):
    accsum = jnp.concatenate(
        [acclo[0] + acclo[1] + h1lo[...], acchi[0] + acchi[1] + h1hi[...]],
        axis=1,
    )
    z = jnp.maximum(dis[...] * accsum + b1[...], 0.0)
    h2 = jnp.dot(z, W2a[...], preferred_element_type=jnp.float32) + jnp.dot(
        xproj[...], W2b[...], preferred_element_type=jnp.float32
    )
    h2p = h2 * dis[...]
    lo_o[...] = h2p[:, :FH]
    hi_o[...] = h2p[:, FH:]


def _tc_post_body(acclo, acchi, h2lo, h2hi, dis, b2, out_o):
    accsum = jnp.concatenate(
        [acclo[0] + acclo[1] + h2lo[...], acchi[0] + acchi[1] + h2hi[...]],
        axis=1,
    )
    out_o[...] = dis[...] * accsum + b2[...]


def _tc_mm(xpad, W1, Wp, bp):
    return pl.pallas_call(
        _tc_mm_body,
        out_shape=(
            jax.ShapeDtypeStruct((NPAD, F), jnp.float32),
            jax.ShapeDtypeStruct((NPAD, 4), jnp.float32),
        ),
    )(xpad, W1, Wp, bp)


def _tc_scale(degpart, h1):
    return pl.pallas_call(
        _tc_scale_body,
        out_shape=(
            jax.ShapeDtypeStruct((NPAD, 1), jnp.float32),
            jax.ShapeDtypeStruct((NPAD, FH), jnp.float32),
            jax.ShapeDtypeStruct((NPAD, FH), jnp.float32),
        ),
    )(degpart, h1)


_BR = 2560


def _tc_mid(acclo, acchi, h1lo, h1hi, dis, b1, xproj, W2a, W2b):
    nblk = NPAD // _BR
    row = lambda i: (i, 0)
    return pl.pallas_call(
        _tc_mid_body,
        grid=(nblk,),
        in_specs=[
            pl.BlockSpec((2, _BR, FH), lambda i: (0, i, 0)),
            pl.BlockSpec((2, _BR, FH), lambda i: (0, i, 0)),
            pl.BlockSpec((_BR, FH), row),
            pl.BlockSpec((_BR, FH), row),
            pl.BlockSpec((_BR, 1), row),
            pl.BlockSpec((1, F), lambda i: (0, 0)),
            pl.BlockSpec((_BR, 4), row),
            pl.BlockSpec((F, F), lambda i: (0, 0)),
            pl.BlockSpec((4, F), lambda i: (0, 0)),
        ],
        out_specs=[
            pl.BlockSpec((_BR, FH), row),
            pl.BlockSpec((_BR, FH), row),
        ],
        out_shape=(
            jax.ShapeDtypeStruct((NPAD, FH), jnp.float32),
            jax.ShapeDtypeStruct((NPAD, FH), jnp.float32),
        ),
    )(acclo, acchi, h1lo, h1hi, dis, b1, xproj, W2a, W2b)


def _tc_post(acclo, acchi, h2lo, h2hi, dis, b2):
    nblk = NPAD // _BR
    row = lambda i: (i, 0)
    return pl.pallas_call(
        _tc_post_body,
        grid=(nblk,),
        in_specs=[
            pl.BlockSpec((2, _BR, FH), lambda i: (0, i, 0)),
            pl.BlockSpec((2, _BR, FH), lambda i: (0, i, 0)),
            pl.BlockSpec((_BR, FH), row),
            pl.BlockSpec((_BR, FH), row),
            pl.BlockSpec((_BR, 1), row),
            pl.BlockSpec((1, F), lambda i: (0, 0)),
        ],
        out_specs=pl.BlockSpec((_BR, F), row),
        out_shape=jax.ShapeDtypeStruct((NPAD, F), jnp.float32),
    )(acclo, acchi, h2lo, h2hi, dis, b2)


def kernel(edge_index, x, W_proj, b_proj, W1, b1, W2, b2):
    pad = jnp.full((EPAD - E,), NPAD - 1, jnp.int32)
    src2 = jnp.concatenate([edge_index[0], pad]).reshape(EPAD // CHUNK, CHUNK)
    dst2 = jnp.concatenate([edge_index[1], pad]).reshape(EPAD // CHUNK, CHUNK)
    xpad = jnp.pad(x, ((0, NPAD - N), (0, 0)))
    ones_hbm = jnp.ones((CHUNK, DEGW), jnp.float32)
    zeros_deg = jnp.zeros((NPAD, DEGW), jnp.float32)
    zeros_acc = jnp.zeros((NPAD, FH), jnp.float32)

    degpart = _deg_call(dst2, ones_hbm, zeros_deg)
    h1, xproj = _tc_mm(xpad, W1, W_proj, b_proj.reshape(1, 4))
    dis, h1lo, h1hi = _tc_scale(degpart, h1)
    acc1lo, acc1hi = _edge_call(src2, dst2, h1lo, h1hi, zeros_acc)
    h2lo, h2hi = _tc_mid(
        acc1lo, acc1hi, h1lo, h1hi, dis, b1.reshape(1, F), xproj, W2[:F], W2[F:]
    )
    acc2lo, acc2hi = _edge_call(src2, dst2, h2lo, h2hi, zeros_acc)
    out = _tc_post(acc2lo, acc2hi, h2lo, h2hi, dis, b2.reshape(1, F))
    return out[:N]
